# baseline (speedup 1.0000x reference)
import functools

import jax
import jax.numpy as jnp
import numpy as np
from jax import lax
from jax.experimental import pallas as pl
from jax.experimental.pallas import tpu as pltpu
from jax.experimental.pallas import tpu_sc as plsc

NC, NS, L = 2, 16, 16
F32 = jnp.float32
_DEC0 = float(np.log(2.0 + 1e-9))
_DEC1 = float(np.log(1.5 + 1e-9))
_BN = 1.0 / float(np.sqrt(1.0 + 1e-5))


def _elu(x):
    return jnp.where(x > 0, x, jnp.exp(jnp.minimum(x, 0.0)) - 1.0)


def _full16(v):
    return jnp.full((16,), v, jnp.int32)


_GDN = lax.GatherDimensionNumbers(
    offset_dims=(), collapsed_slice_dims=(0,), start_index_map=(0,))


def _vtake(v, j):
    return lax.gather(v, jnp.full((16, 1), j, jnp.int32), _GDN, (1,),
                      mode=lax.GatherScatterMode.PROMISE_IN_BOUNDS)


def _make_sc_layer(Ns, Nd, E, H, use_nid):
    CH = 64
    ept = E // NS
    nchunk = ept // CH
    ndpt = Nd // NS
    Hw = 256 // H

    mesh = plsc.VectorSubcoreMesh(
        core_axis_name="c", subcore_axis_name="s", num_cores=NC,
        num_subcores=NS)

    scratch = [
        pltpu.VMEM((2, CH, 256), F32),
        pltpu.VMEM((Ns * 4,), F32),
        pltpu.VMEM((nchunk, CH), jnp.int32),
        pltpu.VMEM((nchunk, CH), jnp.int32),
        pltpu.VMEM((H, ept), F32),
        [pltpu.VMEM((nchunk, CH), jnp.int32) for _ in range(H)],
        pltpu.VMEM((Ns,), jnp.int32) if use_nid else pltpu.VMEM((8,), jnp.int32),
        pltpu.VMEM_SHARED((Nd, 256), F32),
        pltpu.VMEM_SHARED((Nd * H,), F32),
        pltpu.SemaphoreType.DMA,
        pltpu.SemaphoreType.DMA,
    ]

    def body(h_hbm, at_hbm, nid_hbm, src_hbm, dst_hbm, num_hbm, s_hbm,
             rowb, at_v, src_v, dst_v, ebufp, dsth_v, nid_v, out_sp,
             s_sp, sem0, sem1):
        c = lax.axis_index("c")
        t = lax.axis_index("s")

        pltpu.sync_copy(at_hbm.at[pl.ds(c * Ns * 4, Ns * 4)], at_v)
        if use_nid:
            pltpu.sync_copy(nid_hbm.at[c], nid_v)
        pltpu.sync_copy(src_hbm.at[c, pl.ds(t * nchunk, nchunk)], src_v)
        pltpu.sync_copy(dst_hbm.at[c, pl.ds(t * nchunk, nchunk)], dst_v)
        z16 = jnp.zeros((16,), F32)

        def zrow(i, _):
            for kk in range(16):
                rowb[0, i, pl.ds(kk * 16, 16)] = z16
            return 0

        lax.fori_loop(0, CH, zrow, 0)
        for kk in range((ndpt * H) // 16):
            ebufp[0, pl.ds(kk * 16, 16)] = z16
        if ndpt >= CH:
            for zz in range(ndpt // CH):
                pltpu.sync_copy(
                    rowb.at[0],
                    out_sp.at[pl.ds(t * ndpt + zz * CH, CH)])
        else:
            pltpu.sync_copy(rowb.at[0, pl.ds(0, ndpt)],
                            out_sp.at[pl.ds(t * ndpt, ndpt)])
        pltpu.sync_copy(ebufp.at[0, pl.ds(0, ndpt * H)],
                        s_sp.at[pl.ds(t * ndpt * H, ndpt * H)])

        lane = lax.iota(jnp.int32, 16)

        def logit_chunk(j, _):
            for k in range(CH // 16):
                s16 = src_v[j, pl.ds(k * 16, 16)]
                d16 = dst_v[j, pl.ds(k * 16, 16)]
                if use_nid:
                    ns16 = plsc.load_gather(nid_v, [s16])
                    nd16 = plsc.load_gather(nid_v, [d16])
                else:
                    ns16, nd16 = s16, d16
                src_v[j, pl.ds(k * 16, 16)] = ns16 + c * Ns
                ns4 = ns16 * 4
                nd4 = nd16 * 4
                for h in range(H):
                    a_s = plsc.load_gather(at_v, [ns4 + h])
                    a_d = plsc.load_gather(at_v, [nd4 + (2 + h)])
                    al = a_s + a_d
                    al = jnp.where(al > 0, al, 0.2 * al)
                    e16 = jnp.exp(al)
                    ebufp[h, pl.ds(j * CH + k * 16, 16)] = e16
                    dsth_v[h][j, pl.ds(k * 16, 16)] = d16 * H + h
            return 0

        lax.fori_loop(0, nchunk, logit_chunk, 0)
        plsc.subcore_barrier()

        def chunk_body(j, _):
            b = j % 2
            pltpu.make_async_copy(h_hbm.at[src_v.at[j]], rowb.at[b],
                                  sem0).wait()

            @pl.when(j + 1 < nchunk)
            def _():
                pltpu.async_copy(h_hbm.at[src_v.at[j + 1]],
                                 rowb.at[(j + 1) % 2], sem0)

            def sgroup(g, _):
                base = j * CH + g * 16
                evs = [ebufp[h, pl.ds(base, 16)] for h in range(H)]
                for jj in range(16):
                    i = g * 16 + jj
                    for h in range(H):
                        eh = _vtake(evs[h], jj)
                        for kk in range(Hw // 16):
                            off = h * Hw + kk * 16
                            rowb[b, i, pl.ds(off, 16)] = (
                                rowb[b, i, pl.ds(off, 16)] * eh)
                return 0

            lax.fori_loop(0, CH // 16, sgroup, 0)
            pltpu.sync_copy(rowb.at[b], out_sp.at[dst_v.at[j]], add=True)
            for h in range(H):
                pltpu.sync_copy(ebufp.at[h, pl.ds(j * CH, CH)],
                                s_sp.at[dsth_v[h].at[j]], add=True)
            return 0

        pltpu.async_copy(h_hbm.at[src_v.at[0]], rowb.at[0], sem0)
        lax.fori_loop(0, nchunk, chunk_body, 0)

        plsc.subcore_barrier()
        pltpu.sync_copy(out_sp.at[pl.ds(t * ndpt, ndpt)],
                        num_hbm.at[c, pl.ds(t * ndpt, ndpt)])
        pltpu.sync_copy(s_sp.at[pl.ds(t * ndpt * H, ndpt * H)],
                        s_hbm.at[c, pl.ds(t * ndpt * H, ndpt * H)])

    return pl.kernel(
        body,
        out_type=(jax.ShapeDtypeStruct((NC, Nd, 256), F32),
                  jax.ShapeDtypeStruct((NC, Nd * H), F32)),
        mesh=mesh,
        scratch_types=scratch,
        compiler_params=pltpu.CompilerParams(
            needs_layout_passes=False, use_tc_tiling_on_sc=False),
    )


def _tc0_body(feat, w1, acat, f1_o, a1_o):
    f = jnp.dot(feat[...], w1[0], preferred_element_type=F32)
    f1_o[0] = f
    a1_o[0] = jnp.dot(f, acat[0], preferred_element_type=F32)


def _tc0(features, W1s, Acat1):
    nb = 4
    blk = 4096 // nb
    return pl.pallas_call(
        _tc0_body,
        grid=(NC, nb),
        in_specs=[
            pl.BlockSpec((blk, 512), lambda r, b: (b, 0)),
            pl.BlockSpec((1, 512, 256), lambda r, b: (r, 0, 0)),
            pl.BlockSpec((1, 256, 4), lambda r, b: (r, 0, 0)),
        ],
        out_specs=[
            pl.BlockSpec((1, blk, 256), lambda r, b: (r, b, 0)),
            pl.BlockSpec((1, blk, 4), lambda r, b: (r, b, 0)),
        ],
        out_shape=[
            jax.ShapeDtypeStruct((NC, 4096, 256), F32),
            jax.ShapeDtypeStruct((NC, 4096, 4), F32),
        ],
    )(features, W1s, Acat1)


def _head_div(num, s, H):
    Hw = 256 // H
    cols = [num[:, h * Hw:(h + 1) * Hw] /
            (jnp.broadcast_to(s[:, h:h + 1], (num.shape[0], Hw)) + 1e-16)
            for h in range(H)]
    return jnp.concatenate(cols, axis=1) if H > 1 else cols[0]


def _tc1_body(num0, s0, b1, ha0, hb0, bnw, bnb, w2, acat2,
              f2_o, a2_o, zsc_o, zsum_o):
    x = _head_div(num0[0], s0[0], 2) + b1[0]
    g = _elu(x)
    ga = jnp.dot(g, ha0[0], preferred_element_type=F32)[:, 0:1]
    gcol = ga + hb0[pl.program_id(0), 0, 0]
    z = x * gcol
    zsc_o[0] = z * _DEC0
    zsum_o[0] = z
    xb = _elu((x * _BN) * bnw[0] + bnb[0])
    f = jnp.dot(xb, w2[0], preferred_element_type=F32)
    f2_o[0] = f
    a2_o[0] = jnp.dot(f, acat2[0], preferred_element_type=F32)


def _tc1(num0, s0, b1s, ha0s, hb0s, bnws, bnbs, W2s, Acat2):
    return pl.pallas_call(
        _tc1_body,
        grid=(NC,),
        in_specs=[
            pl.BlockSpec((1, 2048, 256), lambda r: (r, 0, 0)),
            pl.BlockSpec((1, 2048, 2), lambda r: (r, 0, 0)),
            pl.BlockSpec((1, 1, 256), lambda r: (r, 0, 0)),
            pl.BlockSpec((1, 256, 8), lambda r: (r, 0, 0)),
            pl.BlockSpec(memory_space=pltpu.SMEM),
            pl.BlockSpec((1, 1, 256), lambda r: (r, 0, 0)),
            pl.BlockSpec((1, 1, 256), lambda r: (r, 0, 0)),
            pl.BlockSpec((1, 256, 256), lambda r: (r, 0, 0)),
            pl.BlockSpec((1, 256, 4), lambda r: (r, 0, 0)),
        ],
        out_specs=[
            pl.BlockSpec((1, 2048, 256), lambda r: (r, 0, 0)),
            pl.BlockSpec((1, 2048, 4), lambda r: (r, 0, 0)),
            pl.BlockSpec((1, 2048, 256), lambda r: (r, 0, 0)),
            pl.BlockSpec((1, 2048, 256), lambda r: (r, 0, 0)),
        ],
        out_shape=[
            jax.ShapeDtypeStruct((NC, 2048, 256), F32),
            jax.ShapeDtypeStruct((NC, 2048, 4), F32),
            jax.ShapeDtypeStruct((NC, 2048, 256), F32),
            jax.ShapeDtypeStruct((NC, 2048, 256), F32),
        ],
    )(num0, s0, b1s, ha0s, hb0s, bnws, bnbs, W2s, Acat2)


def _tc2_body(num1, s1, b2, ha1a, ha1b, hb1, zsc, zsum, bnw, bnb, w3, acat3,
              f3_o, a3_o):
    x = _head_div(num1[0], s1[0], 2) + b2[0]
    ga = (jnp.dot(_elu(x), ha1a[0], preferred_element_type=F32) +
          jnp.dot(_elu(zsc[0]), ha1b[0],
                  preferred_element_type=F32))[:, 0:1]
    gcol = ga + hb1[pl.program_id(0), 0, 0]
    z = x * gcol
    zs1 = zsum[0] + z
    xb = _elu((zs1 * _BN) * bnw[0] + bnb[0])
    f = jnp.dot(xb, w3[0], preferred_element_type=F32)
    f3_o[0] = f
    a3_o[0] = jnp.dot(f, acat3[0], preferred_element_type=F32)


def _tc2(num1, s1, b2s, ha1as, ha1bs, hb1s, zsc0, zsum0, bnws, bnbs, W3s,
         Acat3):
    return pl.pallas_call(
        _tc2_body,
        grid=(NC,),
        in_specs=[
            pl.BlockSpec((1, 1024, 256), lambda r: (r, 0, 0)),
            pl.BlockSpec((1, 1024, 2), lambda r: (r, 0, 0)),
            pl.BlockSpec((1, 1, 256), lambda r: (r, 0, 0)),
            pl.BlockSpec((1, 256, 8), lambda r: (r, 0, 0)),
            pl.BlockSpec((1, 256, 8), lambda r: (r, 0, 0)),
            pl.BlockSpec(memory_space=pltpu.SMEM),
            pl.BlockSpec((1, 1024, 256), lambda r: (r, 0, 0)),
            pl.BlockSpec((1, 1024, 256), lambda r: (r, 0, 0)),
            pl.BlockSpec((1, 1, 256), lambda r: (r, 0, 0)),
            pl.BlockSpec((1, 1, 256), lambda r: (r, 0, 0)),
            pl.BlockSpec((1, 256, 256), lambda r: (r, 0, 0)),
            pl.BlockSpec((1, 256, 4), lambda r: (r, 0, 0)),
        ],
        out_specs=[
            pl.BlockSpec((1, 1024, 256), lambda r: (r, 0, 0)),
            pl.BlockSpec((1, 1024, 4), lambda r: (r, 0, 0)),
        ],
        out_shape=[
            jax.ShapeDtypeStruct((NC, 1024, 256), F32),
            jax.ShapeDtypeStruct((NC, 1024, 4), F32),
        ],
    )(num1, s1, b2s, ha1as, ha1bs, hb1s, zsc0, zsum0, bnws, bnbs, W3s, Acat3)


def _tc3_body(num2, s2, b3, rl, w_om, b_om, u_om, out_o):
    e0 = num2[0] / (jnp.broadcast_to(s2[0], (512, 256)) + 1e-16) + b3[0]
    e1 = num2[1] / (jnp.broadcast_to(s2[1], (512, 256)) + 1e-16) + b3[1]
    xa0 = e0 * rl[0, 0]
    xa1 = e1 * rl[1, 0]
    v0 = jnp.tanh(jnp.dot(xa0, w_om[...], preferred_element_type=F32) +
                  b_om[0])
    v1 = jnp.tanh(jnp.dot(xa1, w_om[...], preferred_element_type=F32) +
                  b_om[0])
    vu0 = jnp.dot(v0, u_om[...], preferred_element_type=F32)[:, 0:1]
    vu1 = jnp.dot(v1, u_om[...], preferred_element_type=F32)[:, 0:1]
    m = jnp.maximum(vu0, vu1)
    x0 = jnp.exp(vu0 - m)
    x1 = jnp.exp(vu1 - m)
    den = x0 + x1
    out_o[...] = xa0 * (x0 / den) + xa1 * (x1 / den)


def _tc3(num2, s2, b3s, RL, w_omega, b_omega, u_omega):
    return pl.pallas_call(
        _tc3_body,
        in_specs=[
            pl.BlockSpec((NC, 512, 256), lambda: (0, 0, 0)),
            pl.BlockSpec((NC, 512, 1), lambda: (0, 0, 0)),
            pl.BlockSpec((NC, 1, 256), lambda: (0, 0, 0)),
            pl.BlockSpec(memory_space=pltpu.SMEM),
            pl.BlockSpec((256, 256), lambda: (0, 0)),
            pl.BlockSpec((1, 256), lambda: (0, 0)),
            pl.BlockSpec((256, 8), lambda: (0, 0)),
        ],
        out_specs=pl.BlockSpec((512, 256), lambda: (0, 0)),
        out_shape=jax.ShapeDtypeStruct((512, 256), F32),
    )(num2, s2, b3s, RL, w_omega, b_omega, u_omega)


def _acat(att_s, att_d, H):
    Hw = 256 // H
    m = jnp.zeros((256, 4), F32)
    for h in range(H):
        m = m.at[h * Hw:(h + 1) * Hw, h].set(att_s[0, h])
        m = m.at[h * Hw:(h + 1) * Hw, 2 + h].set(att_d[0, h])
    return m


_SC_CACHE = {}


def _get_sc(*key):
    if key not in _SC_CACHE:
        _SC_CACHE[key] = _make_sc_layer(*key)
    return _SC_CACHE[key]


def _sc_l0(*a):
    return _get_sc(4096, 2048, 65536, 2, True)(*a)


def _sc_l1(*a):
    return _get_sc(2048, 1024, 32768, 2, False)(*a)


def _sc_l2(*a):
    return _get_sc(1024, 512, 16384, 1, False)(*a)


@jax.jit
def kernel(features, biases_0, biases_1, RL_thresholds, r0_W1, r0_as1, r0_ad1, r0_b1, r0_W2, r0_as2, r0_ad2, r0_b2, r0_W3, r0_as3, r0_ad3, r0_b3, r0_ha0, r0_ha1, r0_hb0, r0_hb1, r0_bnw, r0_bnb, r1_W1, r1_as1, r1_ad1, r1_b1, r1_W2, r1_as2, r1_ad2, r1_b2, r1_W3, r1_as3, r1_ad3, r1_b3, r1_ha0, r1_ha1, r1_hb0, r1_hb1, r1_bnw, r1_bnb, w_omega, b_omega, u_omega, n_ids_0, n_ids_1, ei_r0_l0, ei_r0_l1, ei_r0_l2, ei_r1_l0, ei_r1_l1, ei_r1_l2, batch_nodes):
    W1s = jnp.stack([r0_W1, r1_W1])
    W2s = jnp.stack([r0_W2, r1_W2])
    W3s = jnp.stack([r0_W3, r1_W3])
    Acat1 = jnp.stack([_acat(r0_as1, r0_ad1, 2), _acat(r1_as1, r1_ad1, 2)])
    Acat2 = jnp.stack([_acat(r0_as2, r0_ad2, 2), _acat(r1_as2, r1_ad2, 2)])
    Acat3 = jnp.stack([_acat(r0_as3, r0_ad3, 1), _acat(r1_as3, r1_ad3, 1)])
    b1s = jnp.stack([r0_b1, r1_b1])[:, None, :]
    b2s = jnp.stack([r0_b2, r1_b2])[:, None, :]
    b3s = jnp.stack([r0_b3, r1_b3])[:, None, :]
    def col8(v):
        return jnp.zeros((256, 8), F32).at[:, 0].set(v)

    ha0s = jnp.stack([col8(r0_ha0[0]), col8(r1_ha0[0])])
    ha1as = jnp.stack([col8(r0_ha1[0, :256]), col8(r1_ha1[0, :256])])
    ha1bs = jnp.stack([col8(r0_ha1[0, 256:]), col8(r1_ha1[0, 256:])])
    hb0s = jnp.stack([r0_hb0, r1_hb0])
    hb1s = jnp.stack([r0_hb1, r1_hb1])
    bnws = jnp.stack([r0_bnw, r1_bnw])[:, None, :]
    bnbs = jnp.stack([r0_bnb, r1_bnb])[:, None, :]
    nids = jnp.stack([n_ids_0, n_ids_1])

    def edges(e0, e1):
        src = jnp.stack([e0[0], e1[0]]).reshape(NC, -1, 64)
        dst = jnp.stack([e0[1], e1[1]]).reshape(NC, -1, 64)
        return src, dst

    src0, dst0 = edges(ei_r0_l0, ei_r1_l0)
    src1, dst1 = edges(ei_r0_l1, ei_r1_l1)
    src2, dst2 = edges(ei_r0_l2, ei_r1_l2)

    F1, A1 = _tc0(features, W1s, Acat1)
    num0, s0 = _sc_l0(F1.reshape(NC * 4096, 256), A1.reshape(NC * 4096 * 4),
                      nids, src0, dst0)
    s0 = s0.reshape(NC, 2048, 2)
    F2, A2, zsc0, zsum0 = _tc1(num0, s0, b1s, ha0s, hb0s, bnws, bnbs, W2s,
                               Acat2)
    F2p = jnp.concatenate(
        [F2.reshape(NC * 2048, 256), jnp.zeros((NC * 2048 + 64, 256), F32)])
    num1, s1 = _sc_l1(F2p, A2.reshape(NC * 2048 * 4),
                      jnp.zeros((NC, 8), jnp.int32), src1, dst1)
    s1 = s1.reshape(NC, 1024, 2)
    F3, A3 = _tc2(num1, s1, b2s, ha1as, ha1bs, hb1s, zsc0, zsum0, bnws, bnbs,
                  W3s, Acat3)
    F3p = jnp.concatenate(
        [F3.reshape(NC * 1024, 256), jnp.zeros((NC * 3072 + 64, 256), F32)])
    num2, s2 = _sc_l2(F3p, A3.reshape(NC * 1024 * 4),
                      jnp.zeros((NC, 8), jnp.int32), src2, dst2)
    s2 = s2.reshape(NC, 512, 1)
    return _tc3(num2, s2, b3s, RL_thresholds, w_omega,
                b_omega.reshape(1, 256),
                jnp.zeros((256, 8), F32).at[:, 0].set(u_omega))


# speedup vs baseline: 1.7883x; 1.7883x over previous
import functools

import jax
import jax.numpy as jnp
import numpy as np
from jax import lax
from jax.experimental import pallas as pl
from jax.experimental.pallas import tpu as pltpu
from jax.experimental.pallas import tpu_sc as plsc

NC, NS, L = 2, 16, 16
F32 = jnp.float32
_DEC0 = float(np.log(2.0 + 1e-9))
_DEC1 = float(np.log(1.5 + 1e-9))
_BN = 1.0 / float(np.sqrt(1.0 + 1e-5))


def _elu(x):
    return jnp.where(x > 0, x, jnp.exp(jnp.minimum(x, 0.0)) - 1.0)


def _full16(v):
    return jnp.full((16,), v, jnp.int32)


_GDN = lax.GatherDimensionNumbers(
    offset_dims=(), collapsed_slice_dims=(0,), start_index_map=(0,))


def _vtake(v, j):
    return lax.gather(v, jnp.full((16, 1), j, jnp.int32), _GDN, (1,),
                      mode=lax.GatherScatterMode.PROMISE_IN_BOUNDS)


def _make_sc_layer(Ns, Nd, E, H, use_nid):
    CH = 128
    ept = E // NS
    nchunk = ept // CH
    ndpt = Nd // NS
    Hw = 256 // H

    mesh = plsc.VectorSubcoreMesh(
        core_axis_name="c", subcore_axis_name="s", num_cores=NC,
        num_subcores=NS)

    scratch = [
        pltpu.VMEM((1, CH, 256), F32),
        pltpu.VMEM((Ns * 4,), F32),
        pltpu.VMEM((nchunk, CH), jnp.int32),
        pltpu.VMEM((nchunk, CH), jnp.int32),
        pltpu.VMEM((H, ept), F32),
        [pltpu.VMEM((nchunk, CH), jnp.int32) for _ in range(H)],
        pltpu.VMEM((Ns,), jnp.int32) if use_nid else pltpu.VMEM((8,), jnp.int32),
        pltpu.VMEM_SHARED((Nd, 256), F32),
        pltpu.VMEM_SHARED((Nd * H,), F32),
        pltpu.SemaphoreType.DMA,
        pltpu.SemaphoreType.DMA,
    ]

    def body(h_hbm, at_hbm, nid_hbm, src_hbm, dst_hbm, num_hbm, s_hbm,
             rowb, at_v, src_v, dst_v, ebufp, dsth_v, nid_v, out_sp,
             s_sp, sem0, sem1):
        c = lax.axis_index("c")
        t = lax.axis_index("s")

        pltpu.sync_copy(at_hbm.at[pl.ds(c * Ns * 4, Ns * 4)], at_v)
        if use_nid:
            pltpu.sync_copy(nid_hbm.at[c], nid_v)
        pltpu.sync_copy(src_hbm.at[c, pl.ds(t * nchunk, nchunk)], src_v)
        pltpu.sync_copy(dst_hbm.at[c, pl.ds(t * nchunk, nchunk)], dst_v)
        z16 = jnp.zeros((16,), F32)

        def zrow(i, _):
            for kk in range(16):
                rowb[0, i, pl.ds(kk * 16, 16)] = z16
            return 0

        lax.fori_loop(0, CH, zrow, 0)
        for kk in range((ndpt * H) // 16):
            ebufp[0, pl.ds(kk * 16, 16)] = z16
        if ndpt >= CH:
            for zz in range(ndpt // CH):
                pltpu.sync_copy(
                    rowb.at[0],
                    out_sp.at[pl.ds(t * ndpt + zz * CH, CH)])
        else:
            pltpu.sync_copy(rowb.at[0, pl.ds(0, ndpt)],
                            out_sp.at[pl.ds(t * ndpt, ndpt)])
        pltpu.sync_copy(ebufp.at[0, pl.ds(0, ndpt * H)],
                        s_sp.at[pl.ds(t * ndpt * H, ndpt * H)])

        lane = lax.iota(jnp.int32, 16)

        def logit_chunk(j, _):
            for k in range(CH // 16):
                s16 = src_v[j, pl.ds(k * 16, 16)]
                d16 = dst_v[j, pl.ds(k * 16, 16)]
                if use_nid:
                    ns16 = plsc.load_gather(nid_v, [s16])
                    nd16 = plsc.load_gather(nid_v, [d16])
                else:
                    ns16, nd16 = s16, d16
                src_v[j, pl.ds(k * 16, 16)] = ns16 + c * Ns
                ns4 = ns16 * 4
                nd4 = nd16 * 4
                for h in range(H):
                    a_s = plsc.load_gather(at_v, [ns4 + h])
                    a_d = plsc.load_gather(at_v, [nd4 + (2 + h)])
                    al = a_s + a_d
                    al = jnp.where(al > 0, al, 0.2 * al)
                    e16 = jnp.exp(al)
                    ebufp[h, pl.ds(j * CH + k * 16, 16)] = e16
                    dsth_v[h][j, pl.ds(k * 16, 16)] = d16 * H + h
            return 0

        lax.fori_loop(0, nchunk, logit_chunk, 0)
        plsc.subcore_barrier()

        def chunk_body(j, _):
            b = 0
            pltpu.async_copy(h_hbm.at[src_v.at[j]], rowb.at[b],
                             sem0).wait()

            def sgroup(g, _):
                base = j * CH + g * 16
                evs = [ebufp[h, pl.ds(base, 16)] for h in range(H)]
                for jj in range(16):
                    i = g * 16 + jj
                    for h in range(H):
                        eh = _vtake(evs[h], jj)
                        for kk in range(Hw // 16):
                            off = h * Hw + kk * 16
                            rowb[b, i, pl.ds(off, 16)] = (
                                rowb[b, i, pl.ds(off, 16)] * eh)
                return 0

            lax.fori_loop(0, CH // 16, sgroup, 0)
            pltpu.sync_copy(rowb.at[b], out_sp.at[dst_v.at[j]], add=True)
            for h in range(H):
                pltpu.sync_copy(ebufp.at[h, pl.ds(j * CH, CH)],
                                s_sp.at[dsth_v[h].at[j]], add=True)
            return 0

        lax.fori_loop(0, nchunk, chunk_body, 0)

        plsc.subcore_barrier()
        pltpu.sync_copy(out_sp.at[pl.ds(t * ndpt, ndpt)],
                        num_hbm.at[c, pl.ds(t * ndpt, ndpt)])
        pltpu.sync_copy(s_sp.at[pl.ds(t * ndpt * H, ndpt * H)],
                        s_hbm.at[c, pl.ds(t * ndpt * H, ndpt * H)])

    return pl.kernel(
        body,
        out_type=(jax.ShapeDtypeStruct((NC, Nd, 256), F32),
                  jax.ShapeDtypeStruct((NC, Nd * H), F32)),
        mesh=mesh,
        scratch_types=scratch,
        compiler_params=pltpu.CompilerParams(
            needs_layout_passes=False, use_tc_tiling_on_sc=False),
    )


def _tc0_body(feat, w1, acat, f1_o, a1_o):
    f = jnp.dot(feat[...], w1[0], preferred_element_type=F32)
    f1_o[0] = f
    a1_o[0] = jnp.dot(f, acat[0], preferred_element_type=F32)


def _tc0(features, W1s, Acat1):
    nb = 4
    blk = 4096 // nb
    return pl.pallas_call(
        _tc0_body,
        grid=(NC, nb),
        in_specs=[
            pl.BlockSpec((blk, 512), lambda r, b: (b, 0)),
            pl.BlockSpec((1, 512, 256), lambda r, b: (r, 0, 0)),
            pl.BlockSpec((1, 256, 4), lambda r, b: (r, 0, 0)),
        ],
        out_specs=[
            pl.BlockSpec((1, blk, 256), lambda r, b: (r, b, 0)),
            pl.BlockSpec((1, blk, 4), lambda r, b: (r, b, 0)),
        ],
        out_shape=[
            jax.ShapeDtypeStruct((NC, 4096, 256), F32),
            jax.ShapeDtypeStruct((NC, 4096, 4), F32),
        ],
    )(features, W1s, Acat1)


def _head_div(num, s, H):
    Hw = 256 // H
    cols = [num[:, h * Hw:(h + 1) * Hw] /
            (jnp.broadcast_to(s[:, h:h + 1], (num.shape[0], Hw)) + 1e-16)
            for h in range(H)]
    return jnp.concatenate(cols, axis=1) if H > 1 else cols[0]


def _tc1_body(num0, s0, b1, ha0, hb0, bnw, bnb, w2, acat2,
              f2_o, a2_o, zsc_o, zsum_o):
    x = _head_div(num0[0], s0[0], 2) + b1[0]
    g = _elu(x)
    ga = jnp.dot(g, ha0[0], preferred_element_type=F32)[:, 0:1]
    gcol = ga + hb0[pl.program_id(0), 0, 0]
    z = x * gcol
    zsc_o[0] = z * _DEC0
    zsum_o[0] = z
    xb = _elu((x * _BN) * bnw[0] + bnb[0])
    f = jnp.dot(xb, w2[0], preferred_element_type=F32)
    f2_o[0] = f
    a2_o[0] = jnp.dot(f, acat2[0], preferred_element_type=F32)


def _tc1(num0, s0, b1s, ha0s, hb0s, bnws, bnbs, W2s, Acat2):
    return pl.pallas_call(
        _tc1_body,
        grid=(NC,),
        in_specs=[
            pl.BlockSpec((1, 2048, 256), lambda r: (r, 0, 0)),
            pl.BlockSpec((1, 2048, 2), lambda r: (r, 0, 0)),
            pl.BlockSpec((1, 1, 256), lambda r: (r, 0, 0)),
            pl.BlockSpec((1, 256, 8), lambda r: (r, 0, 0)),
            pl.BlockSpec(memory_space=pltpu.SMEM),
            pl.BlockSpec((1, 1, 256), lambda r: (r, 0, 0)),
            pl.BlockSpec((1, 1, 256), lambda r: (r, 0, 0)),
            pl.BlockSpec((1, 256, 256), lambda r: (r, 0, 0)),
            pl.BlockSpec((1, 256, 4), lambda r: (r, 0, 0)),
        ],
        out_specs=[
            pl.BlockSpec((1, 2048, 256), lambda r: (r, 0, 0)),
            pl.BlockSpec((1, 2048, 4), lambda r: (r, 0, 0)),
            pl.BlockSpec((1, 2048, 256), lambda r: (r, 0, 0)),
            pl.BlockSpec((1, 2048, 256), lambda r: (r, 0, 0)),
        ],
        out_shape=[
            jax.ShapeDtypeStruct((NC, 2048, 256), F32),
            jax.ShapeDtypeStruct((NC, 2048, 4), F32),
            jax.ShapeDtypeStruct((NC, 2048, 256), F32),
            jax.ShapeDtypeStruct((NC, 2048, 256), F32),
        ],
    )(num0, s0, b1s, ha0s, hb0s, bnws, bnbs, W2s, Acat2)


def _tc2_body(num1, s1, b2, ha1a, ha1b, hb1, zsc, zsum, bnw, bnb, w3, acat3,
              f3_o, a3_o):
    x = _head_div(num1[0], s1[0], 2) + b2[0]
    ga = (jnp.dot(_elu(x), ha1a[0], preferred_element_type=F32) +
          jnp.dot(_elu(zsc[0]), ha1b[0],
                  preferred_element_type=F32))[:, 0:1]
    gcol = ga + hb1[pl.program_id(0), 0, 0]
    z = x * gcol
    zs1 = zsum[0] + z
    xb = _elu((zs1 * _BN) * bnw[0] + bnb[0])
    f = jnp.dot(xb, w3[0], preferred_element_type=F32)
    f3_o[0] = f
    a3_o[0] = jnp.dot(f, acat3[0], preferred_element_type=F32)


def _tc2(num1, s1, b2s, ha1as, ha1bs, hb1s, zsc0, zsum0, bnws, bnbs, W3s,
         Acat3):
    return pl.pallas_call(
        _tc2_body,
        grid=(NC,),
        in_specs=[
            pl.BlockSpec((1, 1024, 256), lambda r: (r, 0, 0)),
            pl.BlockSpec((1, 1024, 2), lambda r: (r, 0, 0)),
            pl.BlockSpec((1, 1, 256), lambda r: (r, 0, 0)),
            pl.BlockSpec((1, 256, 8), lambda r: (r, 0, 0)),
            pl.BlockSpec((1, 256, 8), lambda r: (r, 0, 0)),
            pl.BlockSpec(memory_space=pltpu.SMEM),
            pl.BlockSpec((1, 1024, 256), lambda r: (r, 0, 0)),
            pl.BlockSpec((1, 1024, 256), lambda r: (r, 0, 0)),
            pl.BlockSpec((1, 1, 256), lambda r: (r, 0, 0)),
            pl.BlockSpec((1, 1, 256), lambda r: (r, 0, 0)),
            pl.BlockSpec((1, 256, 256), lambda r: (r, 0, 0)),
            pl.BlockSpec((1, 256, 4), lambda r: (r, 0, 0)),
        ],
        out_specs=[
            pl.BlockSpec((1, 1024, 256), lambda r: (r, 0, 0)),
            pl.BlockSpec((1, 1024, 4), lambda r: (r, 0, 0)),
        ],
        out_shape=[
            jax.ShapeDtypeStruct((NC, 1024, 256), F32),
            jax.ShapeDtypeStruct((NC, 1024, 4), F32),
        ],
    )(num1, s1, b2s, ha1as, ha1bs, hb1s, zsc0, zsum0, bnws, bnbs, W3s, Acat3)


def _tc3_body(num2, s2, b3, rl, w_om, b_om, u_om, out_o):
    e0 = num2[0] / (jnp.broadcast_to(s2[0], (512, 256)) + 1e-16) + b3[0]
    e1 = num2[1] / (jnp.broadcast_to(s2[1], (512, 256)) + 1e-16) + b3[1]
    xa0 = e0 * rl[0, 0]
    xa1 = e1 * rl[1, 0]
    v0 = jnp.tanh(jnp.dot(xa0, w_om[...], preferred_element_type=F32) +
                  b_om[0])
    v1 = jnp.tanh(jnp.dot(xa1, w_om[...], preferred_element_type=F32) +
                  b_om[0])
    vu0 = jnp.dot(v0, u_om[...], preferred_element_type=F32)[:, 0:1]
    vu1 = jnp.dot(v1, u_om[...], preferred_element_type=F32)[:, 0:1]
    m = jnp.maximum(vu0, vu1)
    x0 = jnp.exp(vu0 - m)
    x1 = jnp.exp(vu1 - m)
    den = x0 + x1
    out_o[...] = xa0 * (x0 / den) + xa1 * (x1 / den)


def _tc3(num2, s2, b3s, RL, w_omega, b_omega, u_omega):
    return pl.pallas_call(
        _tc3_body,
        in_specs=[
            pl.BlockSpec((NC, 512, 256), lambda: (0, 0, 0)),
            pl.BlockSpec((NC, 512, 1), lambda: (0, 0, 0)),
            pl.BlockSpec((NC, 1, 256), lambda: (0, 0, 0)),
            pl.BlockSpec(memory_space=pltpu.SMEM),
            pl.BlockSpec((256, 256), lambda: (0, 0)),
            pl.BlockSpec((1, 256), lambda: (0, 0)),
            pl.BlockSpec((256, 8), lambda: (0, 0)),
        ],
        out_specs=pl.BlockSpec((512, 256), lambda: (0, 0)),
        out_shape=jax.ShapeDtypeStruct((512, 256), F32),
    )(num2, s2, b3s, RL, w_omega, b_omega, u_omega)


def _acat(att_s, att_d, H):
    Hw = 256 // H
    m = jnp.zeros((256, 4), F32)
    for h in range(H):
        m = m.at[h * Hw:(h + 1) * Hw, h].set(att_s[0, h])
        m = m.at[h * Hw:(h + 1) * Hw, 2 + h].set(att_d[0, h])
    return m


_SC_CACHE = {}


def _get_sc(*key):
    if key not in _SC_CACHE:
        _SC_CACHE[key] = _make_sc_layer(*key)
    return _SC_CACHE[key]


def _sc_l0(*a):
    return _get_sc(4096, 2048, 65536, 2, True)(*a)


def _sc_l1(*a):
    return _get_sc(2048, 1024, 32768, 2, False)(*a)


def _sc_l2(*a):
    return _get_sc(1024, 512, 16384, 1, False)(*a)


@jax.jit
def kernel(features, biases_0, biases_1, RL_thresholds, r0_W1, r0_as1, r0_ad1, r0_b1, r0_W2, r0_as2, r0_ad2, r0_b2, r0_W3, r0_as3, r0_ad3, r0_b3, r0_ha0, r0_ha1, r0_hb0, r0_hb1, r0_bnw, r0_bnb, r1_W1, r1_as1, r1_ad1, r1_b1, r1_W2, r1_as2, r1_ad2, r1_b2, r1_W3, r1_as3, r1_ad3, r1_b3, r1_ha0, r1_ha1, r1_hb0, r1_hb1, r1_bnw, r1_bnb, w_omega, b_omega, u_omega, n_ids_0, n_ids_1, ei_r0_l0, ei_r0_l1, ei_r0_l2, ei_r1_l0, ei_r1_l1, ei_r1_l2, batch_nodes):
    W1s = jnp.stack([r0_W1, r1_W1])
    W2s = jnp.stack([r0_W2, r1_W2])
    W3s = jnp.stack([r0_W3, r1_W3])
    Acat1 = jnp.stack([_acat(r0_as1, r0_ad1, 2), _acat(r1_as1, r1_ad1, 2)])
    Acat2 = jnp.stack([_acat(r0_as2, r0_ad2, 2), _acat(r1_as2, r1_ad2, 2)])
    Acat3 = jnp.stack([_acat(r0_as3, r0_ad3, 1), _acat(r1_as3, r1_ad3, 1)])
    b1s = jnp.stack([r0_b1, r1_b1])[:, None, :]
    b2s = jnp.stack([r0_b2, r1_b2])[:, None, :]
    b3s = jnp.stack([r0_b3, r1_b3])[:, None, :]
    def col8(v):
        return jnp.zeros((256, 8), F32).at[:, 0].set(v)

    ha0s = jnp.stack([col8(r0_ha0[0]), col8(r1_ha0[0])])
    ha1as = jnp.stack([col8(r0_ha1[0, :256]), col8(r1_ha1[0, :256])])
    ha1bs = jnp.stack([col8(r0_ha1[0, 256:]), col8(r1_ha1[0, 256:])])
    hb0s = jnp.stack([r0_hb0, r1_hb0])
    hb1s = jnp.stack([r0_hb1, r1_hb1])
    bnws = jnp.stack([r0_bnw, r1_bnw])[:, None, :]
    bnbs = jnp.stack([r0_bnb, r1_bnb])[:, None, :]
    nids = jnp.stack([n_ids_0, n_ids_1])

    def edges(e0, e1):
        src = jnp.stack([e0[0], e1[0]]).reshape(NC, -1, 128)
        dst = jnp.stack([e0[1], e1[1]]).reshape(NC, -1, 128)
        return src, dst

    src0, dst0 = edges(ei_r0_l0, ei_r1_l0)
    src1, dst1 = edges(ei_r0_l1, ei_r1_l1)
    src2, dst2 = edges(ei_r0_l2, ei_r1_l2)

    F1, A1 = _tc0(features, W1s, Acat1)
    num0, s0 = _sc_l0(F1.reshape(NC * 4096, 256), A1.reshape(NC * 4096 * 4),
                      nids, src0, dst0)
    s0 = s0.reshape(NC, 2048, 2)
    F2, A2, zsc0, zsum0 = _tc1(num0, s0, b1s, ha0s, hb0s, bnws, bnbs, W2s,
                               Acat2)
    F2p = jnp.concatenate(
        [F2.reshape(NC * 2048, 256), jnp.zeros((NC * 2048 + 64, 256), F32)])
    num1, s1 = _sc_l1(F2p, A2.reshape(NC * 2048 * 4),
                      jnp.zeros((NC, 8), jnp.int32), src1, dst1)
    s1 = s1.reshape(NC, 1024, 2)
    F3, A3 = _tc2(num1, s1, b2s, ha1as, ha1bs, hb1s, zsc0, zsum0, bnws, bnbs,
                  W3s, Acat3)
    F3p = jnp.concatenate(
        [F3.reshape(NC * 1024, 256), jnp.zeros((NC * 3072 + 64, 256), F32)])
    num2, s2 = _sc_l2(F3p, A3.reshape(NC * 1024 * 4),
                      jnp.zeros((NC, 8), jnp.int32), src2, dst2)
    s2 = s2.reshape(NC, 512, 1)
    return _tc3(num2, s2, b3s, RL_thresholds, w_omega,
                b_omega.reshape(1, 256),
                jnp.zeros((256, 8), F32).at[:, 0].set(u_omega))


# speedup vs baseline: 1.8784x; 1.0504x over previous
import functools

import jax
import jax.numpy as jnp
import numpy as np
from jax import lax
from jax.experimental import pallas as pl
from jax.experimental.pallas import tpu as pltpu
from jax.experimental.pallas import tpu_sc as plsc

NC, NS, L = 2, 16, 16
F32 = jnp.float32
_DEC0 = float(np.log(2.0 + 1e-9))
_DEC1 = float(np.log(1.5 + 1e-9))
_BN = 1.0 / float(np.sqrt(1.0 + 1e-5))


def _elu(x):
    return jnp.where(x > 0, x, jnp.exp(jnp.minimum(x, 0.0)) - 1.0)


def _full16(v):
    return jnp.full((16,), v, jnp.int32)


_GDN = lax.GatherDimensionNumbers(
    offset_dims=(), collapsed_slice_dims=(0,), start_index_map=(0,))


def _vtake(v, j):
    return lax.gather(v, jnp.full((16, 1), j, jnp.int32), _GDN, (1,),
                      mode=lax.GatherScatterMode.PROMISE_IN_BOUNDS)


def _make_sc_layer(Ns, Nd, E, H, use_nid):
    CH = 128
    ept = E // NS
    nchunk = ept // CH
    ndpt = Nd // NS
    Hw = 256 // H

    mesh = plsc.VectorSubcoreMesh(
        core_axis_name="c", subcore_axis_name="s", num_cores=NC,
        num_subcores=NS)

    scratch = [
        pltpu.VMEM((1, CH, 256), F32),
        pltpu.VMEM((Ns * 4,), F32),
        pltpu.VMEM((nchunk, CH), jnp.int32),
        pltpu.VMEM((nchunk, CH), jnp.int32),
        pltpu.VMEM((H, ept), F32),
        [pltpu.VMEM((nchunk, CH), jnp.int32) for _ in range(H)],
        pltpu.VMEM((Ns,), jnp.int32) if use_nid else pltpu.VMEM((8,), jnp.int32),
        pltpu.VMEM_SHARED((Nd, 256), F32),
        pltpu.VMEM_SHARED((Nd * H,), F32),
        pltpu.SemaphoreType.DMA,
        pltpu.SemaphoreType.DMA,
    ]

    def body(h_hbm, at_hbm, nid_hbm, src_hbm, dst_hbm, num_hbm, s_hbm,
             rowb, at_v, src_v, dst_v, ebufp, dsth_v, nid_v, out_sp,
             s_sp, sem0, sem1):
        c = lax.axis_index("c")
        t = lax.axis_index("s")

        pltpu.sync_copy(at_hbm.at[pl.ds(c * Ns * 4, Ns * 4)], at_v)
        if use_nid:
            pltpu.sync_copy(nid_hbm.at[c], nid_v)
        pltpu.sync_copy(src_hbm.at[c, pl.ds(t * nchunk, nchunk)], src_v)
        pltpu.sync_copy(dst_hbm.at[c, pl.ds(t * nchunk, nchunk)], dst_v)
        z16 = jnp.zeros((16,), F32)

        def zrow(i, _):
            for kk in range(16):
                rowb[0, i, pl.ds(kk * 16, 16)] = z16
            return 0

        lax.fori_loop(0, CH, zrow, 0)
        for kk in range((ndpt * H) // 16):
            ebufp[0, pl.ds(kk * 16, 16)] = z16
        if ndpt >= CH:
            for zz in range(ndpt // CH):
                pltpu.sync_copy(
                    rowb.at[0],
                    out_sp.at[pl.ds(t * ndpt + zz * CH, CH)])
        else:
            pltpu.sync_copy(rowb.at[0, pl.ds(0, ndpt)],
                            out_sp.at[pl.ds(t * ndpt, ndpt)])
        pltpu.sync_copy(ebufp.at[0, pl.ds(0, ndpt * H)],
                        s_sp.at[pl.ds(t * ndpt * H, ndpt * H)])

        lane = lax.iota(jnp.int32, 16)

        def logit_chunk(j, _):
            for k in range(CH // 16):
                s16 = src_v[j, pl.ds(k * 16, 16)]
                d16 = dst_v[j, pl.ds(k * 16, 16)]
                if use_nid:
                    ns16 = plsc.load_gather(nid_v, [s16])
                    nd16 = plsc.load_gather(nid_v, [d16])
                else:
                    ns16, nd16 = s16, d16
                src_v[j, pl.ds(k * 16, 16)] = ns16 + c * Ns
                ns4 = ns16 * 4
                nd4 = nd16 * 4
                for h in range(H):
                    a_s = plsc.load_gather(at_v, [ns4 + h])
                    a_d = plsc.load_gather(at_v, [nd4 + (2 + h)])
                    al = a_s + a_d
                    al = jnp.where(al > 0, al, 0.2 * al)
                    e16 = jnp.exp(al)
                    ebufp[h, pl.ds(j * CH + k * 16, 16)] = e16
                    dsth_v[h][j, pl.ds(k * 16, 16)] = d16 * H + h
            return 0

        lax.fori_loop(0, nchunk, logit_chunk, 0)
        plsc.subcore_barrier()

        def chunk_body(j, _):
            b = 0
            pltpu.async_copy(h_hbm.at[src_v.at[j]], rowb.at[b],
                             sem0).wait()

            def sgroup(g, _):
                base = j * CH + g * 16
                evs = [ebufp[h, pl.ds(base, 16)] for h in range(H)]
                for jj in range(16):
                    i = g * 16 + jj
                    for h in range(H):
                        eh = _vtake(evs[h], jj)
                        for kk in range(Hw // 16):
                            off = h * Hw + kk * 16
                            rowb[b, i, pl.ds(off, 16)] = (
                                rowb[b, i, pl.ds(off, 16)] * eh)
                return 0

            lax.fori_loop(0, CH // 16, sgroup, 0)
            pltpu.sync_copy(rowb.at[b], out_sp.at[dst_v.at[j]], add=True)
            for h in range(H):
                pltpu.sync_copy(ebufp.at[h, pl.ds(j * CH, CH)],
                                s_sp.at[dsth_v[h].at[j]], add=True)
            return 0

        lax.fori_loop(0, nchunk, chunk_body, 0)

        plsc.subcore_barrier()
        pltpu.sync_copy(out_sp.at[pl.ds(t * ndpt, ndpt)],
                        num_hbm.at[c, pl.ds(t * ndpt, ndpt)])
        pltpu.sync_copy(s_sp.at[pl.ds(t * ndpt * H, ndpt * H)],
                        s_hbm.at[c, pl.ds(t * ndpt * H, ndpt * H)])

    return pl.kernel(
        body,
        out_type=(jax.ShapeDtypeStruct((NC, Nd, 256), F32),
                  jax.ShapeDtypeStruct((NC, Nd * H), F32)),
        mesh=mesh,
        scratch_types=scratch,
        compiler_params=pltpu.CompilerParams(
            needs_layout_passes=False, use_tc_tiling_on_sc=False),
    )


def _tc0_body(feat, w1, acat, f1_o, a1_o):
    f = jnp.dot(feat[...], w1[0], preferred_element_type=F32)
    f1_o[0] = f
    a1_o[0] = jnp.dot(f, acat[0], preferred_element_type=F32)


def _tc0(features, W1s, Acat1):
    nb = 4
    blk = 4096 // nb
    return pl.pallas_call(
        _tc0_body,
        grid=(NC, nb),
        in_specs=[
            pl.BlockSpec((blk, 512), lambda r, b: (b, 0)),
            pl.BlockSpec((1, 512, 256), lambda r, b: (r, 0, 0)),
            pl.BlockSpec((1, 256, 4), lambda r, b: (r, 0, 0)),
        ],
        out_specs=[
            pl.BlockSpec((1, blk, 256), lambda r, b: (r, b, 0)),
            pl.BlockSpec((1, blk, 4), lambda r, b: (r, b, 0)),
        ],
        out_shape=[
            jax.ShapeDtypeStruct((NC, 4096, 256), F32),
            jax.ShapeDtypeStruct((NC, 4096, 4), F32),
        ],
    )(features, W1s, Acat1)


def _head_div(num, s, H):
    Hw = 256 // H
    cols = [num[:, h * Hw:(h + 1) * Hw] /
            (jnp.broadcast_to(s[:, h:h + 1], (num.shape[0], Hw)) + 1e-16)
            for h in range(H)]
    return jnp.concatenate(cols, axis=1) if H > 1 else cols[0]


def _tc1_body(num0, s0, b1, ha0, hb0, bnw, bnb, w2, acat2,
              f2_o, a2_o, zsc_o, zsum_o):
    x = _head_div(num0[0], s0[0], 2) + b1[0]
    g = _elu(x)
    ga = jnp.dot(g, ha0[0], preferred_element_type=F32)[:, 0:1]
    gcol = ga + hb0[pl.program_id(0), 0, 0]
    z = x * gcol
    zsc_o[0] = z * _DEC0
    zsum_o[0] = z
    xb = _elu((x * _BN) * bnw[0] + bnb[0])
    f = jnp.dot(xb, w2[0], preferred_element_type=F32)
    f2_o[0] = f
    a2_o[0] = jnp.dot(f, acat2[0], preferred_element_type=F32)


def _tc1(num0, s0, b1s, ha0s, hb0s, bnws, bnbs, W2s, Acat2):
    return pl.pallas_call(
        _tc1_body,
        grid=(NC,),
        in_specs=[
            pl.BlockSpec((1, 2048, 256), lambda r: (r, 0, 0)),
            pl.BlockSpec((1, 2048, 2), lambda r: (r, 0, 0)),
            pl.BlockSpec((1, 1, 256), lambda r: (r, 0, 0)),
            pl.BlockSpec((1, 256, 8), lambda r: (r, 0, 0)),
            pl.BlockSpec(memory_space=pltpu.SMEM),
            pl.BlockSpec((1, 1, 256), lambda r: (r, 0, 0)),
            pl.BlockSpec((1, 1, 256), lambda r: (r, 0, 0)),
            pl.BlockSpec((1, 256, 256), lambda r: (r, 0, 0)),
            pl.BlockSpec((1, 256, 4), lambda r: (r, 0, 0)),
        ],
        out_specs=[
            pl.BlockSpec((1, 2048, 256), lambda r: (r, 0, 0)),
            pl.BlockSpec((1, 2048, 4), lambda r: (r, 0, 0)),
            pl.BlockSpec((1, 2048, 256), lambda r: (r, 0, 0)),
            pl.BlockSpec((1, 2048, 256), lambda r: (r, 0, 0)),
        ],
        out_shape=[
            jax.ShapeDtypeStruct((NC, 2048, 256), F32),
            jax.ShapeDtypeStruct((NC, 2048, 4), F32),
            jax.ShapeDtypeStruct((NC, 2048, 256), F32),
            jax.ShapeDtypeStruct((NC, 2048, 256), F32),
        ],
    )(num0, s0, b1s, ha0s, hb0s, bnws, bnbs, W2s, Acat2)


def _tc2_body(num1, s1, b2, ha1a, ha1b, hb1, zsc, zsum, bnw, bnb, w3, acat3,
              f3_o, a3_o):
    x = _head_div(num1[0], s1[0], 2) + b2[0]
    ga = (jnp.dot(_elu(x), ha1a[0], preferred_element_type=F32) +
          jnp.dot(_elu(zsc[0]), ha1b[0],
                  preferred_element_type=F32))[:, 0:1]
    gcol = ga + hb1[pl.program_id(0), 0, 0]
    z = x * gcol
    zs1 = zsum[0] + z
    xb = _elu((zs1 * _BN) * bnw[0] + bnb[0])
    f = jnp.dot(xb, w3[0], preferred_element_type=F32)
    f3_o[0] = f
    a3_o[0] = jnp.dot(f, acat3[0], preferred_element_type=F32)


def _tc2(num1, s1, b2s, ha1as, ha1bs, hb1s, zsc0, zsum0, bnws, bnbs, W3s,
         Acat3):
    return pl.pallas_call(
        _tc2_body,
        grid=(NC,),
        in_specs=[
            pl.BlockSpec((1, 1024, 256), lambda r: (r, 0, 0)),
            pl.BlockSpec((1, 1024, 2), lambda r: (r, 0, 0)),
            pl.BlockSpec((1, 1, 256), lambda r: (r, 0, 0)),
            pl.BlockSpec((1, 256, 8), lambda r: (r, 0, 0)),
            pl.BlockSpec((1, 256, 8), lambda r: (r, 0, 0)),
            pl.BlockSpec(memory_space=pltpu.SMEM),
            pl.BlockSpec((1, 1024, 256), lambda r: (r, 0, 0)),
            pl.BlockSpec((1, 1024, 256), lambda r: (r, 0, 0)),
            pl.BlockSpec((1, 1, 256), lambda r: (r, 0, 0)),
            pl.BlockSpec((1, 1, 256), lambda r: (r, 0, 0)),
            pl.BlockSpec((1, 256, 256), lambda r: (r, 0, 0)),
            pl.BlockSpec((1, 256, 4), lambda r: (r, 0, 0)),
        ],
        out_specs=[
            pl.BlockSpec((1, 1024, 256), lambda r: (r, 0, 0)),
            pl.BlockSpec((1, 1024, 4), lambda r: (r, 0, 0)),
        ],
        out_shape=[
            jax.ShapeDtypeStruct((NC, 1024, 256), F32),
            jax.ShapeDtypeStruct((NC, 1024, 4), F32),
        ],
    )(num1, s1, b2s, ha1as, ha1bs, hb1s, zsc0, zsum0, bnws, bnbs, W3s, Acat3)


def _tc3_body(num2, s2, b3, rl, w_om, b_om, u_om, out_o):
    e0 = num2[0] / (jnp.broadcast_to(s2[0], (512, 256)) + 1e-16) + b3[0]
    e1 = num2[1] / (jnp.broadcast_to(s2[1], (512, 256)) + 1e-16) + b3[1]
    xa0 = e0 * rl[0, 0]
    xa1 = e1 * rl[1, 0]
    v0 = jnp.tanh(jnp.dot(xa0, w_om[...], preferred_element_type=F32) +
                  b_om[0])
    v1 = jnp.tanh(jnp.dot(xa1, w_om[...], preferred_element_type=F32) +
                  b_om[0])
    vu0 = jnp.dot(v0, u_om[...], preferred_element_type=F32)[:, 0:1]
    vu1 = jnp.dot(v1, u_om[...], preferred_element_type=F32)[:, 0:1]
    m = jnp.maximum(vu0, vu1)
    x0 = jnp.exp(vu0 - m)
    x1 = jnp.exp(vu1 - m)
    den = x0 + x1
    out_o[...] = xa0 * (x0 / den) + xa1 * (x1 / den)


def _tc3(num2, s2, b3s, RL, w_omega, b_omega, u_omega):
    return pl.pallas_call(
        _tc3_body,
        in_specs=[
            pl.BlockSpec((NC, 512, 256), lambda: (0, 0, 0)),
            pl.BlockSpec((NC, 512, 1), lambda: (0, 0, 0)),
            pl.BlockSpec((NC, 1, 256), lambda: (0, 0, 0)),
            pl.BlockSpec(memory_space=pltpu.SMEM),
            pl.BlockSpec((256, 256), lambda: (0, 0)),
            pl.BlockSpec((1, 256), lambda: (0, 0)),
            pl.BlockSpec((256, 8), lambda: (0, 0)),
        ],
        out_specs=pl.BlockSpec((512, 256), lambda: (0, 0)),
        out_shape=jax.ShapeDtypeStruct((512, 256), F32),
    )(num2, s2, b3s, RL, w_omega, b_omega, u_omega)


def _acat(att_s, att_d, H):
    Hw = 256 // H
    m = jnp.zeros((256, 4), F32)
    for h in range(H):
        m = m.at[h * Hw:(h + 1) * Hw, h].set(att_s[0, h])
        m = m.at[h * Hw:(h + 1) * Hw, 2 + h].set(att_d[0, h])
    return m


_SC_CACHE = {}


def _get_sc(*key):
    if key not in _SC_CACHE:
        _SC_CACHE[key] = _make_sc_layer(*key)
    return _SC_CACHE[key]


def _sc_l0(*a):
    return _get_sc(4096, 2048, 65536, 2, True)(*a)


def _sc_l1(*a):
    return _get_sc(2048, 1024, 32768, 2, False)(*a)


def _sc_l2(*a):
    return _get_sc(1024, 512, 16384, 1, False)(*a)


@jax.jit
def kernel(features, biases_0, biases_1, RL_thresholds, r0_W1, r0_as1, r0_ad1, r0_b1, r0_W2, r0_as2, r0_ad2, r0_b2, r0_W3, r0_as3, r0_ad3, r0_b3, r0_ha0, r0_ha1, r0_hb0, r0_hb1, r0_bnw, r0_bnb, r1_W1, r1_as1, r1_ad1, r1_b1, r1_W2, r1_as2, r1_ad2, r1_b2, r1_W3, r1_as3, r1_ad3, r1_b3, r1_ha0, r1_ha1, r1_hb0, r1_hb1, r1_bnw, r1_bnb, w_omega, b_omega, u_omega, n_ids_0, n_ids_1, ei_r0_l0, ei_r0_l1, ei_r0_l2, ei_r1_l0, ei_r1_l1, ei_r1_l2, batch_nodes):
    W1s = jnp.stack([r0_W1, r1_W1])
    W2s = jnp.stack([r0_W2, r1_W2])
    W3s = jnp.stack([r0_W3, r1_W3])
    Acat1 = jnp.stack([_acat(r0_as1, r0_ad1, 2), _acat(r1_as1, r1_ad1, 2)])
    Acat2 = jnp.stack([_acat(r0_as2, r0_ad2, 2), _acat(r1_as2, r1_ad2, 2)])
    Acat3 = jnp.stack([_acat(r0_as3, r0_ad3, 1), _acat(r1_as3, r1_ad3, 1)])
    b1s = jnp.stack([r0_b1, r1_b1])[:, None, :]
    b2s = jnp.stack([r0_b2, r1_b2])[:, None, :]
    b3s = jnp.stack([r0_b3, r1_b3])[:, None, :]
    def col8(v):
        return jnp.zeros((256, 8), F32).at[:, 0].set(v)

    ha0s = jnp.stack([col8(r0_ha0[0]), col8(r1_ha0[0])])
    ha1as = jnp.stack([col8(r0_ha1[0, :256]), col8(r1_ha1[0, :256])])
    ha1bs = jnp.stack([col8(r0_ha1[0, 256:]), col8(r1_ha1[0, 256:])])
    hb0s = jnp.stack([r0_hb0, r1_hb0])
    hb1s = jnp.stack([r0_hb1, r1_hb1])
    bnws = jnp.stack([r0_bnw, r1_bnw])[:, None, :]
    bnbs = jnp.stack([r0_bnb, r1_bnb])[:, None, :]
    nids = jnp.stack([n_ids_0, n_ids_1])

    def edges(e0, e1):
        src = jnp.stack([e0[0], e1[0]]).reshape(NC, -1, 128)
        dst = jnp.stack([e0[1], e1[1]]).reshape(NC, -1, 128)
        return src, dst

    src0, dst0 = edges(ei_r0_l0, ei_r1_l0)
    src1, dst1 = edges(ei_r0_l1, ei_r1_l1)
    src2, dst2 = edges(ei_r0_l2, ei_r1_l2)

    F1, A1 = _tc0(features, W1s, Acat1)
    num0, s0 = _sc_l0(F1.reshape(NC * 4096, 256), A1.reshape(NC * 4096 * 4),
                      nids, src0, dst0)
    s0 = s0.reshape(NC, 2048, 2)
    F2, A2, zsc0, zsum0 = _tc1(num0, s0, b1s, ha0s, hb0s, bnws, bnbs, W2s,
                               Acat2)
    num1, s1 = _sc_l1(F2.reshape(NC * 2048, 256), A2.reshape(NC * 2048 * 4),
                      jnp.zeros((NC, 8), jnp.int32), src1, dst1)
    s1 = s1.reshape(NC, 1024, 2)
    F3, A3 = _tc2(num1, s1, b2s, ha1as, ha1bs, hb1s, zsc0, zsum0, bnws, bnbs,
                  W3s, Acat3)
    num2, s2 = _sc_l2(F3.reshape(NC * 1024, 256), A3.reshape(NC * 1024 * 4),
                      jnp.zeros((NC, 8), jnp.int32), src2, dst2)
    s2 = s2.reshape(NC, 512, 1)
    return _tc3(num2, s2, b3s, RL_thresholds, w_omega,
                b_omega.reshape(1, 256),
                jnp.zeros((256, 8), F32).at[:, 0].set(u_omega))
